# Initial kernel scaffold; baseline (speedup 1.0000x reference)
import jax
import jax.numpy as jnp
from jax.experimental import pallas as pl


def kernel(x, knot_y, W, b):
    raise NotImplementedError("write your pallas kernel here")


# speedup vs baseline: 2300.2322x; 2300.2322x over previous
import functools

import jax
import jax.numpy as jnp
from jax import lax
from jax.experimental import pallas as pl
from jax.experimental.pallas import tpu as pltpu
from jax.experimental.pallas import tpu_sc as plsc

B = 16384
IN_F = 128
OUT_F = 128
KNOTS = 50
X_MIN = -10.0
X_MAX = 10.0
H = (X_MAX - X_MIN) / (KNOTS - 1)
INV_H = 1.0 / H

NUM_CORES = 2
NUM_SUBCORES = 16
LANES = 16
NW = NUM_CORES * NUM_SUBCORES

TOTAL = B * IN_F
CHUNK = TOTAL // NW
ROWS_PER_TILE = CHUNK // IN_F
VECS_PER_ROW = IN_F // LANES


def _spline_body(x_hbm, ky_hbm, out_hbm, xbuf, kybuf):
    wid = lax.axis_index("s") * NUM_CORES + lax.axis_index("c")
    base = wid * CHUNK
    pltpu.sync_copy(ky_hbm, kybuf)
    pltpu.sync_copy(x_hbm.at[pl.ds(base, CHUNK)], xbuf)

    iota = lax.iota(jnp.int32, LANES)
    col_base = [(iota + j * LANES) * KNOTS for j in range(VECS_PER_ROW)]

    def row_body(r, carry):
        off = r * IN_F
        for j in range(VECS_PER_ROW):
            xv = xbuf[pl.ds(off + j * LANES, LANES)]
            u = (xv - X_MIN) * INV_H
            uc = jnp.minimum(jnp.maximum(u, 1.0), 47.0)
            idx = uc.astype(jnp.int32)
            idxf = idx.astype(jnp.float32)
            t = (xv - (idxf * H + X_MIN)) * INV_H
            gbase = col_base[j] + idx
            y0 = plsc.load_gather(kybuf, [gbase - 1])
            y1 = plsc.load_gather(kybuf, [gbase])
            y2 = plsc.load_gather(kybuf, [gbase + 1])
            y3 = plsc.load_gather(kybuf, [gbase + 2])
            c1 = y2 - y0
            c2 = 2.0 * y0 - 5.0 * y1 + 4.0 * y2 - y3
            c3 = (y3 - y0) + 3.0 * (y1 - y2)
            xbuf[pl.ds(off + j * LANES, LANES)] = (
                y1 + 0.5 * t * (c1 + t * (c2 + t * c3)))
        return carry

    lax.fori_loop(0, ROWS_PER_TILE, row_body, 0)
    pltpu.sync_copy(xbuf, out_hbm.at[pl.ds(base, CHUNK)])


_spline_sc = functools.partial(
    pl.kernel,
    mesh=plsc.VectorSubcoreMesh(core_axis_name="c", subcore_axis_name="s"),
    out_type=jax.ShapeDtypeStruct((TOTAL,), jnp.float32),
    scratch_types=[
        pltpu.VMEM((CHUNK,), jnp.float32),
        pltpu.VMEM((IN_F * KNOTS,), jnp.float32),
    ],
    compiler_params=pltpu.CompilerParams(needs_layout_passes=False),
)(_spline_body)


BT = 1024


def _mm_body(t_ref, wt_ref, b_ref, o_ref):
    o_ref[...] = jnp.dot(
        t_ref[...], wt_ref[...], preferred_element_type=jnp.float32
    ) + b_ref[...]


_mm = pl.pallas_call(
    _mm_body,
    grid=(B // BT,),
    in_specs=[
        pl.BlockSpec((BT, IN_F), lambda i: (i, 0)),
        pl.BlockSpec((IN_F, OUT_F), lambda i: (0, 0)),
        pl.BlockSpec((1, OUT_F), lambda i: (0, 0)),
    ],
    out_specs=pl.BlockSpec((BT, OUT_F), lambda i: (i, 0)),
    out_shape=jax.ShapeDtypeStruct((B, OUT_F), jnp.float32),
)


def kernel(x, knot_y, W, b):
    transformed = _spline_sc(x.reshape(-1), knot_y.reshape(-1))
    return _mm(transformed.reshape(B, IN_F), W.T, b.reshape(1, OUT_F))


# speedup vs baseline: 3840.7815x; 1.6697x over previous
import functools

import jax
import jax.numpy as jnp
from jax import lax
from jax.experimental import pallas as pl
from jax.experimental.pallas import tpu as pltpu
from jax.experimental.pallas import tpu_sc as plsc

B = 16384
IN_F = 128
OUT_F = 128
KNOTS = 50
X_MIN = -10.0
X_MAX = 10.0
H = (X_MAX - X_MIN) / (KNOTS - 1)
INV_H = 1.0 / H

NUM_CORES = 2
NUM_SUBCORES = 16
LANES = 16
NW = NUM_CORES * NUM_SUBCORES

TOTAL = B * IN_F
CHUNK = TOTAL // NW
ROWS_PER_TILE = CHUNK // IN_F
VECS_PER_ROW = IN_F // LANES


NSUB = 2
SUB = CHUNK // NSUB
ROWS_SUB = SUB // IN_F


def _spline_body(x_hbm, ky_hbm, out_hbm, xbuf, obuf, kybuf):
    wid = lax.axis_index("s") * NUM_CORES + lax.axis_index("c")
    base = wid * CHUNK
    pltpu.sync_copy(ky_hbm, kybuf)

    iota = lax.iota(jnp.int32, LANES)
    col_base = [(iota + j * LANES) * KNOTS for j in range(VECS_PER_ROW)]

    for sub in range(NSUB):
        off_h = base + sub * SUB
        pltpu.sync_copy(x_hbm.at[pl.ds(off_h, SUB)], xbuf)

        @plsc.parallel_loop(0, ROWS_SUB, 1)
        def row_body(r):
            off = r * IN_F
            for j in range(VECS_PER_ROW):
                xv = xbuf[pl.ds(off + j * LANES, LANES)]
                u = (xv - X_MIN) * INV_H
                uc = jnp.minimum(jnp.maximum(u, 1.0), 47.0)
                idx = uc.astype(jnp.int32)
                t = u - idx.astype(jnp.float32)
                gbase = col_base[j] + idx
                y0 = plsc.load_gather(kybuf, [gbase - 1])
                y1 = plsc.load_gather(kybuf, [gbase])
                y2 = plsc.load_gather(kybuf, [gbase + 1])
                y3 = plsc.load_gather(kybuf, [gbase + 2])
                c1 = y2 - y0
                c3 = (y3 - y0) + 3.0 * (y1 - y2)
                c2 = 2.0 * (y2 - y1) - c1 - c3
                obuf[pl.ds(off + j * LANES, LANES)] = (
                    y1 + 0.5 * t * (c1 + t * (c2 + t * c3)))

        pltpu.sync_copy(obuf, out_hbm.at[pl.ds(off_h, SUB)])


_spline_sc = functools.partial(
    pl.kernel,
    mesh=plsc.VectorSubcoreMesh(core_axis_name="c", subcore_axis_name="s"),
    out_type=jax.ShapeDtypeStruct((TOTAL,), jnp.float32),
    scratch_types=[
        pltpu.VMEM((SUB,), jnp.float32),
        pltpu.VMEM((SUB,), jnp.float32),
        pltpu.VMEM((IN_F * KNOTS,), jnp.float32),
    ],
    compiler_params=pltpu.CompilerParams(needs_layout_passes=False),
)(_spline_body)


BT = 1024


def _mm_body(t_ref, wt_ref, b_ref, o_ref):
    o_ref[...] = jnp.dot(
        t_ref[...], wt_ref[...], preferred_element_type=jnp.float32
    ) + b_ref[...]


_mm = pl.pallas_call(
    _mm_body,
    grid=(B // BT,),
    in_specs=[
        pl.BlockSpec((BT, IN_F), lambda i: (i, 0)),
        pl.BlockSpec((IN_F, OUT_F), lambda i: (0, 0)),
        pl.BlockSpec((1, OUT_F), lambda i: (0, 0)),
    ],
    out_specs=pl.BlockSpec((BT, OUT_F), lambda i: (i, 0)),
    out_shape=jax.ShapeDtypeStruct((B, OUT_F), jnp.float32),
)


def kernel(x, knot_y, W, b):
    transformed = _spline_sc(x.reshape(-1), knot_y.reshape(-1))
    return _mm(transformed.reshape(B, IN_F), W.T, b.reshape(1, OUT_F))


# speedup vs baseline: 3944.9989x; 1.0271x over previous
import functools

import jax
import jax.numpy as jnp
from jax import lax
from jax.experimental import pallas as pl
from jax.experimental.pallas import tpu as pltpu
from jax.experimental.pallas import tpu_sc as plsc

B = 16384
IN_F = 128
OUT_F = 128
KNOTS = 50
NINT = 48
X_MIN = -10.0
X_MAX = 10.0
H = (X_MAX - X_MIN) / (KNOTS - 1)
INV_H = 1.0 / H
U0 = -X_MIN * INV_H

NUM_CORES = 2
NUM_SUBCORES = 16
LANES = 16
NW = NUM_CORES * NUM_SUBCORES

TOTAL = B * IN_F
CHUNK = TOTAL // NW
VECS_PER_ROW = IN_F // LANES

NSUB = 2
SUB = CHUNK // NSUB
ROWS_SUB = SUB // IN_F


def _spline_body(x_hbm, ky_hbm, out_hbm, xbuf, obuf, kybuf, ca, cb, cc, cd):
    wid = lax.axis_index("s") * NUM_CORES + lax.axis_index("c")
    base = wid * CHUNK
    pltpu.sync_copy(ky_hbm, kybuf)

    iota = lax.iota(jnp.int32, LANES)

    @plsc.parallel_loop(0, IN_F, 1)
    def build(f):
        for jj in range(NINT // LANES):
            g0 = f * KNOTS + jj * LANES + iota
            g0 = jnp.minimum(g0, IN_F * KNOTS - 4)
            y0 = plsc.load_gather(kybuf, [g0])
            y1 = plsc.load_gather(kybuf, [g0 + 1])
            y2 = plsc.load_gather(kybuf, [g0 + 2])
            y3 = plsc.load_gather(kybuf, [g0 + 3])
            bv = 0.5 * (y2 - y0)
            dv = 0.5 * (y3 - y0) + 1.5 * (y1 - y2)
            cv = (y2 - y1) - bv - dv
            sl = pl.ds(f * NINT + jj * LANES, LANES)
            ca[sl] = y1
            cb[sl] = bv
            cc[sl] = cv
            cd[sl] = dv

    col_base = [(iota + j * LANES) * NINT - 1 for j in range(VECS_PER_ROW)]

    for sub in range(NSUB):
        off_h = base + sub * SUB
        pltpu.sync_copy(x_hbm.at[pl.ds(off_h, SUB)], xbuf)

        @plsc.parallel_loop(0, ROWS_SUB, 1)
        def row_body(r):
            off = r * IN_F
            for j in range(VECS_PER_ROW):
                xv = xbuf[pl.ds(off + j * LANES, LANES)]
                u = xv * INV_H + U0
                uc = jnp.minimum(jnp.maximum(u, 1.0), 47.0)
                idx = uc.astype(jnp.int32)
                t = u - idx.astype(jnp.float32)
                g = col_base[j] + idx
                av = plsc.load_gather(ca, [g])
                bv = plsc.load_gather(cb, [g])
                cv = plsc.load_gather(cc, [g])
                dv = plsc.load_gather(cd, [g])
                obuf[pl.ds(off + j * LANES, LANES)] = (
                    ((dv * t + cv) * t + bv) * t + av)

        pltpu.sync_copy(obuf, out_hbm.at[pl.ds(off_h, SUB)])


_spline_sc = functools.partial(
    pl.kernel,
    mesh=plsc.VectorSubcoreMesh(core_axis_name="c", subcore_axis_name="s"),
    out_type=jax.ShapeDtypeStruct((TOTAL,), jnp.float32),
    scratch_types=[
        pltpu.VMEM((SUB,), jnp.float32),
        pltpu.VMEM((SUB,), jnp.float32),
        pltpu.VMEM((IN_F * KNOTS,), jnp.float32),
        pltpu.VMEM((IN_F * NINT,), jnp.float32),
        pltpu.VMEM((IN_F * NINT,), jnp.float32),
        pltpu.VMEM((IN_F * NINT,), jnp.float32),
        pltpu.VMEM((IN_F * NINT,), jnp.float32),
    ],
    compiler_params=pltpu.CompilerParams(needs_layout_passes=False),
)(_spline_body)


BT = 1024


def _mm_body(t_ref, wt_ref, b_ref, o_ref):
    o_ref[...] = jnp.dot(
        t_ref[...], wt_ref[...], preferred_element_type=jnp.float32
    ) + b_ref[...]


_mm = pl.pallas_call(
    _mm_body,
    grid=(B // BT,),
    in_specs=[
        pl.BlockSpec((BT, IN_F), lambda i: (i, 0)),
        pl.BlockSpec((IN_F, OUT_F), lambda i: (0, 0)),
        pl.BlockSpec((1, OUT_F), lambda i: (0, 0)),
    ],
    out_specs=pl.BlockSpec((BT, OUT_F), lambda i: (i, 0)),
    out_shape=jax.ShapeDtypeStruct((B, OUT_F), jnp.float32),
)


def kernel(x, knot_y, W, b):
    transformed = _spline_sc(x.reshape(-1), knot_y.reshape(-1))
    return _mm(transformed.reshape(B, IN_F), W.T, b.reshape(1, OUT_F))


# speedup vs baseline: 4111.9178x; 1.0423x over previous
import functools

import jax
import jax.numpy as jnp
from jax import lax
from jax.experimental import pallas as pl
from jax.experimental.pallas import tpu as pltpu
from jax.experimental.pallas import tpu_sc as plsc

B = 16384
IN_F = 128
OUT_F = 128
KNOTS = 50
NINT = 48
X_MIN = -10.0
X_MAX = 10.0
H = (X_MAX - X_MIN) / (KNOTS - 1)
INV_H = 1.0 / H
U0 = -X_MIN * INV_H

NUM_CORES = 2
NUM_SUBCORES = 16
LANES = 16
NW = NUM_CORES * NUM_SUBCORES

TOTAL = B * IN_F
CHUNK = TOTAL // NW
VECS_PER_ROW = IN_F // LANES

NSUB = 4
SUB = CHUNK // NSUB
ROWS_SUB = SUB // IN_F


def _spline_body(x_hbm, ky_hbm, out_hbm,
                 xb0, xb1, ob0, ob1, kybuf, ca, cb, cc, cd,
                 sky, si0, si1, so0, so1):
    wid = lax.axis_index("s") * NUM_CORES + lax.axis_index("c")
    base = wid * CHUNK
    xb, ob, si, so = [xb0, xb1], [ob0, ob1], [si0, si1], [so0, so1]

    cky = pltpu.async_copy(ky_hbm, kybuf, sky)
    cin = [pltpu.async_copy(x_hbm.at[pl.ds(base, SUB)], xb[0], si[0]), None]
    cky.wait()

    iota = lax.iota(jnp.int32, LANES)

    @plsc.parallel_loop(0, IN_F, 1)
    def build(f):
        for jj in range(NINT // LANES):
            g0 = f * KNOTS + jj * LANES + iota
            g0 = jnp.minimum(g0, IN_F * KNOTS - 4)
            y0 = plsc.load_gather(kybuf, [g0])
            y1 = plsc.load_gather(kybuf, [g0 + 1])
            y2 = plsc.load_gather(kybuf, [g0 + 2])
            y3 = plsc.load_gather(kybuf, [g0 + 3])
            bv = 0.5 * (y2 - y0)
            dv = 0.5 * (y3 - y0) + 1.5 * (y1 - y2)
            cv = (y2 - y1) - bv - dv
            sl = pl.ds(f * NINT + jj * LANES, LANES)
            ca[sl] = y1
            cb[sl] = bv
            cc[sl] = cv
            cd[sl] = dv

    col_base = [(iota + j * LANES) * NINT - 1 for j in range(VECS_PER_ROW)]

    cout = [None, None]
    for sub in range(NSUB):
        cur = sub % 2
        nxt = (sub + 1) % 2
        if sub + 1 < NSUB:
            cin[nxt] = pltpu.async_copy(
                x_hbm.at[pl.ds(base + (sub + 1) * SUB, SUB)], xb[nxt], si[nxt])
        cin[cur].wait()
        if cout[cur] is not None:
            cout[cur].wait()
        xbuf = xb[cur]
        obuf = ob[cur]

        @plsc.parallel_loop(0, ROWS_SUB, 1)
        def row_body(r):
            off = r * IN_F
            for j in range(VECS_PER_ROW):
                xv = xbuf[pl.ds(off + j * LANES, LANES)]
                u = xv * INV_H + U0
                uc = jnp.minimum(jnp.maximum(u, 1.0), 47.0)
                idx = uc.astype(jnp.int32)
                t = u - idx.astype(jnp.float32)
                g = col_base[j] + idx
                av = plsc.load_gather(ca, [g])
                bv = plsc.load_gather(cb, [g])
                cv = plsc.load_gather(cc, [g])
                dv = plsc.load_gather(cd, [g])
                obuf[pl.ds(off + j * LANES, LANES)] = (
                    ((dv * t + cv) * t + bv) * t + av)

        cout[cur] = pltpu.async_copy(
            obuf, out_hbm.at[pl.ds(base + sub * SUB, SUB)], so[cur])
    for c in cout:
        if c is not None:
            c.wait()


_spline_sc = functools.partial(
    pl.kernel,
    mesh=plsc.VectorSubcoreMesh(core_axis_name="c", subcore_axis_name="s"),
    out_type=jax.ShapeDtypeStruct((TOTAL,), jnp.float32),
    scratch_types=[
        pltpu.VMEM((SUB,), jnp.float32),
        pltpu.VMEM((SUB,), jnp.float32),
        pltpu.VMEM((SUB,), jnp.float32),
        pltpu.VMEM((SUB,), jnp.float32),
        pltpu.VMEM((IN_F * KNOTS,), jnp.float32),
        pltpu.VMEM((IN_F * NINT,), jnp.float32),
        pltpu.VMEM((IN_F * NINT,), jnp.float32),
        pltpu.VMEM((IN_F * NINT,), jnp.float32),
        pltpu.VMEM((IN_F * NINT,), jnp.float32),
        pltpu.SemaphoreType.DMA,
        pltpu.SemaphoreType.DMA,
        pltpu.SemaphoreType.DMA,
        pltpu.SemaphoreType.DMA,
        pltpu.SemaphoreType.DMA,
    ],
    compiler_params=pltpu.CompilerParams(needs_layout_passes=False),
)(_spline_body)


BT = 1024


def _mm_body(t_ref, wt_ref, b_ref, o_ref):
    o_ref[...] = jnp.dot(
        t_ref[...], wt_ref[...], preferred_element_type=jnp.float32
    ) + b_ref[...]


_mm = pl.pallas_call(
    _mm_body,
    grid=(B // BT,),
    in_specs=[
        pl.BlockSpec((BT, IN_F), lambda i: (i, 0)),
        pl.BlockSpec((IN_F, OUT_F), lambda i: (0, 0)),
        pl.BlockSpec((1, OUT_F), lambda i: (0, 0)),
    ],
    out_specs=pl.BlockSpec((BT, OUT_F), lambda i: (i, 0)),
    out_shape=jax.ShapeDtypeStruct((B, OUT_F), jnp.float32),
)


def kernel(x, knot_y, W, b):
    transformed = _spline_sc(x.reshape(-1), knot_y.reshape(-1))
    return _mm(transformed.reshape(B, IN_F), W.T, b.reshape(1, OUT_F))
